# Initial kernel scaffold; baseline (speedup 1.0000x reference)
import jax
import jax.numpy as jnp
from jax.experimental import pallas as pl


def kernel(x, t, idx):
    raise NotImplementedError("write your pallas kernel here")


# speedup vs baseline: 1.0106x; 1.0106x over previous
import jax
import jax.numpy as jnp
from jax import lax
from jax.experimental import pallas as pl
from jax.experimental.pallas import tpu as pltpu
from jax.experimental.pallas import tpu_sc as plsc

N_CORES = 2
N_SUBCORES = 16
N_WORKERS = N_CORES * N_SUBCORES


def _gather_body(x_hbm, t_hbm, idx_hbm, out_x_hbm, out_t_hbm,
                 idx_v, rows_x, rows_t, sem_x, sem_t):
    b_per_w = idx_v.shape[0]
    wid = lax.axis_index("s") * N_CORES + lax.axis_index("c")
    base = wid * b_per_w
    pltpu.sync_copy(idx_hbm.at[pl.ds(base, b_per_w)], idx_v)
    cx = pltpu.async_copy(x_hbm.at[idx_v], rows_x, sem_x)
    ct = pltpu.async_copy(t_hbm.at[idx_v], rows_t, sem_t)
    cx.wait()
    ct.wait()
    pltpu.sync_copy(rows_x, out_x_hbm.at[pl.ds(base, b_per_w)])
    pltpu.sync_copy(rows_t, out_t_hbm.at[pl.ds(base, b_per_w)])


def kernel(x, t, idx):
    n = idx.shape[0]
    b_per_w = -(-n // (8 * N_WORKERS)) * 8
    n_pad = b_per_w * N_WORKERS
    idx32 = idx.astype(jnp.int32)
    if n_pad != n:
        idx32 = jnp.concatenate(
            [idx32, jnp.zeros((n_pad - n,), dtype=jnp.int32)])
    x_flat = x.reshape(-1)
    t_flat = t.reshape(-1)

    mesh = plsc.VectorSubcoreMesh(
        core_axis_name="c", subcore_axis_name="s",
        num_cores=N_CORES, num_subcores=N_SUBCORES)
    out_x, out_t = pl.kernel(
        _gather_body,
        out_type=(
            jax.ShapeDtypeStruct((n_pad,), jnp.float32),
            jax.ShapeDtypeStruct((n_pad,), jnp.float32),
        ),
        mesh=mesh,
        scratch_types=[
            pltpu.VMEM((b_per_w,), jnp.int32),
            pltpu.VMEM((b_per_w,), jnp.float32),
            pltpu.VMEM((b_per_w,), jnp.float32),
            pltpu.SemaphoreType.DMA,
            pltpu.SemaphoreType.DMA,
        ],
        name="holdout_sampler_gather",
    )(x_flat, t_flat, idx32)

    return (out_x[:n].reshape(n, 1), out_t[:n].reshape(n, 1))


# speedup vs baseline: 2.5620x; 2.5351x over previous
import jax
import jax.numpy as jnp
from jax import lax
from jax.experimental import pallas as pl
from jax.experimental.pallas import tpu as pltpu
from jax.experimental.pallas import tpu_sc as plsc

N_CORES = 2
N_SUBCORES = 16
N_WORKERS = N_CORES * N_SUBCORES


def _gather_body(x_hbm, t_hbm, idx_hbm, out_x_hbm, out_t_hbm,
                 idx_v, rows_x, rows_t, sem_x, sem_t):
    b_per_w = idx_v.shape[0]
    wid = lax.axis_index("s") * N_CORES + lax.axis_index("c")
    base = wid * b_per_w
    pltpu.sync_copy(idx_hbm.at[pl.ds(base, b_per_w)], idx_v)
    cx = pltpu.async_copy(x_hbm.at[idx_v], rows_x, sem_x)
    ct = pltpu.async_copy(t_hbm.at[idx_v], rows_t, sem_t)
    cx.wait()
    ct.wait()
    pltpu.sync_copy(rows_x, out_x_hbm.at[pl.ds(base, b_per_w)])
    pltpu.sync_copy(rows_t, out_t_hbm.at[pl.ds(base, b_per_w)])


def _flatten_padded(a):
    n_rows = a.shape[0]
    n_pad = -(-n_rows // 1024) * 1024
    if n_pad != n_rows:
        a = jnp.pad(a, ((0, n_pad - n_rows), (0, 0)))
    return a.reshape(-1)


def kernel(x, t, idx):
    n = idx.shape[0]
    b_per_w = -(-n // (8 * N_WORKERS)) * 8
    n_pad = b_per_w * N_WORKERS
    idx32 = idx.astype(jnp.int32)
    if n_pad != n:
        idx32 = jnp.concatenate(
            [idx32, jnp.zeros((n_pad - n,), dtype=jnp.int32)])
    x_flat = _flatten_padded(x)
    t_flat = _flatten_padded(t)

    mesh = plsc.VectorSubcoreMesh(
        core_axis_name="c", subcore_axis_name="s",
        num_cores=N_CORES, num_subcores=N_SUBCORES)
    out_x, out_t = pl.kernel(
        _gather_body,
        out_type=(
            jax.ShapeDtypeStruct((n_pad,), jnp.float32),
            jax.ShapeDtypeStruct((n_pad,), jnp.float32),
        ),
        mesh=mesh,
        scratch_types=[
            pltpu.VMEM((b_per_w,), jnp.int32),
            pltpu.VMEM((b_per_w,), jnp.float32),
            pltpu.VMEM((b_per_w,), jnp.float32),
            pltpu.SemaphoreType.DMA,
            pltpu.SemaphoreType.DMA,
        ],
        name="holdout_sampler_gather",
    )(x_flat, t_flat, idx32)

    return (out_x[:n].reshape(n, 1), out_t[:n].reshape(n, 1))


# speedup vs baseline: 2.6413x; 1.0310x over previous
import functools

import jax
import jax.numpy as jnp
from jax import lax
from jax.experimental import pallas as pl
from jax.experimental.pallas import tpu as pltpu
from jax.experimental.pallas import tpu_sc as plsc

N_CORES = 2
N_SUBCORES = 16
N_WORKERS = N_CORES * N_SUBCORES


def _gather_body(table_hbm, idx_hbm, out_hbm, idx_v, rows_v, sem):
    b_per_w = idx_v.shape[0]
    wid = lax.axis_index("s") * N_CORES + lax.axis_index("c")
    base = wid * b_per_w
    pltpu.sync_copy(idx_hbm.at[pl.ds(base, b_per_w)], idx_v)
    pltpu.async_copy(table_hbm.at[idx_v], rows_v, sem).wait()
    pltpu.sync_copy(rows_v, out_hbm.at[pl.ds(base, b_per_w)])


def _flatten_padded(a):
    n_rows = a.shape[0]
    n_pad = -(-n_rows // 1024) * 1024
    if n_pad != n_rows:
        a = jnp.pad(a, ((0, n_pad - n_rows), (0, 0)))
    return a.reshape(-1)


def _make_gather(n_pad, b_per_w, name):
    mesh = plsc.VectorSubcoreMesh(
        core_axis_name="c", subcore_axis_name="s",
        num_cores=N_CORES, num_subcores=N_SUBCORES)
    return pl.kernel(
        _gather_body,
        out_type=jax.ShapeDtypeStruct((n_pad,), jnp.float32),
        mesh=mesh,
        scratch_types=[
            pltpu.VMEM((b_per_w,), jnp.int32),
            pltpu.VMEM((b_per_w,), jnp.float32),
            pltpu.SemaphoreType.DMA,
        ],
        name=name,
    )


def kernel(x, t, idx):
    n = idx.shape[0]
    b_per_w = -(-n // (8 * N_WORKERS)) * 8
    n_pad = b_per_w * N_WORKERS
    idx32 = idx.astype(jnp.int32)
    if n_pad != n:
        idx32 = jnp.concatenate(
            [idx32, jnp.zeros((n_pad - n,), dtype=jnp.int32)])
    x_flat = _flatten_padded(x)
    t_flat = _flatten_padded(t)

    out_x = _make_gather(n_pad, b_per_w, "holdout_gather_x")(x_flat, idx32)
    out_t = _make_gather(n_pad, b_per_w, "holdout_gather_t")(t_flat, idx32)

    return (out_x[:n].reshape(n, 1), out_t[:n].reshape(n, 1))
